# baseline (device time: 537053 ns/iter reference)
import jax
import jax.numpy as jnp
from jax import lax
from jax.experimental import pallas as pl
from jax.experimental.pallas import tpu as pltpu

K = 4


def kernel(x):
    m, n = x.shape
    half = n // 2
    out_m = 2 * m
    rows_c = m // K

    def body(x_ref, out_ref, send_buf, local_sem, stage_sems, send_sems,
             recv_sems):
        my_x = lax.axis_index("x")
        my_y = lax.axis_index("y")
        my_z = lax.axis_index("z")
        partner = (1 - my_x, my_y, my_z)

        barrier = pltpu.get_barrier_semaphore()
        pl.semaphore_signal(
            barrier, inc=1, device_id=partner,
            device_id_type=pl.DeviceIdType.MESH,
        )
        pl.semaphore_wait(barrier, 1)

        local_copy = pltpu.make_async_copy(
            x_ref.at[:, pl.ds(my_x * half, half)],
            out_ref.at[pl.ds(my_x * m, m), :],
            local_sem,
        )
        local_copy.start()

        def stage(k):
            return pltpu.make_async_copy(
                x_ref.at[pl.ds(k * rows_c, rows_c),
                         pl.ds((1 - my_x) * half, half)],
                send_buf.at[pl.ds(k * rows_c, rows_c), :],
                stage_sems.at[k],
            )

        def rdma(k):
            return pltpu.make_async_remote_copy(
                src_ref=send_buf.at[pl.ds(k * rows_c, rows_c), :],
                dst_ref=out_ref.at[pl.ds(my_x * m + k * rows_c, rows_c), :],
                send_sem=send_sems.at[k],
                recv_sem=recv_sems.at[k],
                device_id=partner,
                device_id_type=pl.DeviceIdType.MESH,
            )

        stage(0).start()
        for k in range(K):
            stage(k).wait()
            if k + 1 < K:
                stage(k + 1).start()
            rdma(k).start()

        local_copy.wait()
        for k in range(K):
            rdma(k).wait_send()
        for k in range(K):
            rdma(k).wait_recv()

    return pl.pallas_call(
        body,
        out_shape=jax.ShapeDtypeStruct((out_m, half), x.dtype),
        in_specs=[pl.BlockSpec(memory_space=pl.ANY)],
        out_specs=pl.BlockSpec(memory_space=pl.ANY),
        scratch_shapes=[
            pltpu.VMEM((m, half), x.dtype),
            pltpu.SemaphoreType.DMA,
            pltpu.SemaphoreType.DMA((K,)),
            pltpu.SemaphoreType.DMA((K,)),
            pltpu.SemaphoreType.DMA((K,)),
        ],
        compiler_params=pltpu.CompilerParams(collective_id=0),
    )(x)


# device time: 214023 ns/iter; 2.5093x vs baseline; 2.5093x over previous
import jax
import jax.numpy as jnp
from jax import lax
from jax.experimental import pallas as pl
from jax.experimental.pallas import tpu as pltpu

K = 4


def kernel(x):
    m, n = x.shape
    half = n // 2
    out_m = 2 * m
    rows_c = m // K

    def body(x_ref, out_ref, send_buf, keep_buf, keep_sem, keep_out_sem,
             stage_sems, send_sems, recv_sems):
        my_x = lax.axis_index("x")
        my_y = lax.axis_index("y")
        my_z = lax.axis_index("z")
        partner = (1 - my_x, my_y, my_z)

        barrier = pltpu.get_barrier_semaphore()
        pl.semaphore_signal(
            barrier, inc=1, device_id=partner,
            device_id_type=pl.DeviceIdType.MESH,
        )
        pl.semaphore_wait(barrier, 1)

        keep_in = pltpu.make_async_copy(
            x_ref.at[:, pl.ds(my_x * half, half)], keep_buf, keep_sem,
        )
        keep_in.start()

        def stage(k):
            return pltpu.make_async_copy(
                x_ref.at[pl.ds(k * rows_c, rows_c),
                         pl.ds((1 - my_x) * half, half)],
                send_buf.at[pl.ds(k * rows_c, rows_c), :],
                stage_sems.at[k],
            )

        def rdma(k):
            return pltpu.make_async_remote_copy(
                src_ref=send_buf.at[pl.ds(k * rows_c, rows_c), :],
                dst_ref=out_ref.at[pl.ds(my_x * m + k * rows_c, rows_c), :],
                send_sem=send_sems.at[k],
                recv_sem=recv_sems.at[k],
                device_id=partner,
                device_id_type=pl.DeviceIdType.MESH,
            )

        stage(0).start()
        for k in range(K):
            stage(k).wait()
            if k + 1 < K:
                stage(k + 1).start()
            rdma(k).start()

        keep_in.wait()
        keep_out = pltpu.make_async_copy(
            keep_buf, out_ref.at[pl.ds(my_x * m, m), :], keep_out_sem,
        )
        keep_out.start()
        keep_out.wait()

        for k in range(K):
            rdma(k).wait_send()
        for k in range(K):
            rdma(k).wait_recv()

    return pl.pallas_call(
        body,
        out_shape=jax.ShapeDtypeStruct((out_m, half), x.dtype),
        in_specs=[pl.BlockSpec(memory_space=pl.ANY)],
        out_specs=pl.BlockSpec(memory_space=pl.ANY),
        scratch_shapes=[
            pltpu.VMEM((m, half), x.dtype),
            pltpu.VMEM((m, half), x.dtype),
            pltpu.SemaphoreType.DMA,
            pltpu.SemaphoreType.DMA,
            pltpu.SemaphoreType.DMA((K,)),
            pltpu.SemaphoreType.DMA((K,)),
            pltpu.SemaphoreType.DMA((K,)),
        ],
        compiler_params=pltpu.CompilerParams(collective_id=0),
    )(x)


# device time: 106017 ns/iter; 5.0657x vs baseline; 2.0188x over previous
import jax
import jax.numpy as jnp
from jax import lax
from jax.experimental import pallas as pl
from jax.experimental.pallas import tpu as pltpu

RQ = 1024
CQ = 4
RC = RQ // CQ
A = 344
B = 336
C2 = RQ - A - B


def kernel(x):
    m, n = x.shape
    half = n // 2
    out_m = 2 * m

    def body(x_ref, out_ref, sbuf, xrbuf, yrbuf, zrbuf, keep_buf,
             keep_sem, keep_out_sem, stage_sems,
             xsend, xrecv, ysend, yrecv, zsend, zrecv, csem):
        my_x = lax.axis_index("x")
        my_y = lax.axis_index("y")
        my_z = lax.axis_index("z")
        xp = (1 - my_x, my_y, my_z)
        yp = (my_x, 1 - my_y, my_z)
        zp = (my_x, my_y, 1 - my_z)
        q = 2 * my_y + my_z
        qy = 2 * (1 - my_y) + my_z
        qz = 2 * my_y + (1 - my_z)
        qd = 2 * (1 - my_y) + (1 - my_z)

        ocol = (1 - my_x) * half
        obase = my_x * m
        rbase = (1 - my_x) * m

        def rdma(src, dst, ssem, rsem, dev):
            return pltpu.make_async_remote_copy(
                src_ref=src, dst_ref=dst, send_sem=ssem, recv_sem=rsem,
                device_id=dev, device_id_type=pl.DeviceIdType.MESH,
            )

        barrier = pltpu.get_barrier_semaphore()
        for nb in (xp, yp, zp):
            pl.semaphore_signal(
                barrier, inc=1, device_id=nb,
                device_id_type=pl.DeviceIdType.MESH,
            )
        pl.semaphore_wait(barrier, 3)

        keep_in = pltpu.make_async_copy(
            x_ref.at[:, pl.ds(my_x * half, half)], keep_buf, keep_sem,
        )
        keep_in.start()

        st0 = pltpu.make_async_copy(
            x_ref.at[pl.ds(q * RQ, RQ), pl.ds(ocol, half)],
            sbuf.at[pl.ds(0, RQ), :], stage_sems.at[0],
        )
        st1 = pltpu.make_async_copy(
            x_ref.at[pl.ds(qd * RQ, A), pl.ds(ocol, half)],
            sbuf.at[pl.ds(RQ, A), :], stage_sems.at[1],
        )
        st0.start()
        st1.start()
        st0.wait()

        xd = []
        for k in range(CQ):
            d = rdma(sbuf.at[pl.ds(k * RC, RC), :],
                     xrbuf.at[pl.ds(k * RC, RC), :],
                     xsend.at[k], xrecv.at[k], xp)
            d.start()
            xd.append(d)
        st1.wait()
        xe = rdma(sbuf.at[pl.ds(RQ, A), :],
                  out_ref.at[pl.ds(obase + qd * RQ, A), :],
                  xsend.at[CQ], xrecv.at[CQ], xp)
        xe.start()

        yd = []
        zd = []
        copies = []
        for k in range(CQ):
            xd[k].wait_recv()
            dy = rdma(xrbuf.at[pl.ds(k * RC, RC), :],
                      yrbuf.at[pl.ds(k * RC, RC), :],
                      ysend.at[k], yrecv.at[k], yp)
            dy.start()
            yd.append(dy)
            dz = rdma(xrbuf.at[pl.ds(k * RC, RC), :],
                      zrbuf.at[pl.ds(k * RC, RC), :],
                      zsend.at[k], zrecv.at[k], zp)
            dz.start()
            zd.append(dz)
            cp = pltpu.make_async_copy(
                xrbuf.at[pl.ds(k * RC, RC), :],
                out_ref.at[pl.ds(rbase + q * RQ + k * RC, RC), :],
                csem.at[k],
            )
            cp.start()
            copies.append(cp)

        for k in range(3):
            zd[k].wait_recv()
            cp = pltpu.make_async_copy(
                zrbuf.at[pl.ds(k * RC, RC), :],
                out_ref.at[pl.ds(rbase + qz * RQ + k * RC, RC), :],
                csem.at[CQ + k],
            )
            cp.start()
            copies.append(cp)
        y2 = rdma(zrbuf.at[pl.ds(A, B), :],
                  out_ref.at[pl.ds(rbase + qz * RQ + A, B), :],
                  ysend.at[CQ], yrecv.at[CQ], yp)
        y2.start()

        for k in range(CQ):
            yd[k].wait_recv()
            cp = pltpu.make_async_copy(
                yrbuf.at[pl.ds(k * RC, RC), :],
                out_ref.at[pl.ds(rbase + qy * RQ + k * RC, RC), :],
                csem.at[2 * CQ + k],
            )
            cp.start()
            copies.append(cp)
        z2 = rdma(yrbuf.at[pl.ds(A + B, C2), :],
                  out_ref.at[pl.ds(rbase + qy * RQ + A + B, C2), :],
                  zsend.at[CQ], zrecv.at[CQ], zp)
        z2.start()

        zd[3].wait_recv()
        cp = pltpu.make_async_copy(
            zrbuf.at[pl.ds(3 * RC, RC), :],
            out_ref.at[pl.ds(rbase + qz * RQ + 3 * RC, RC), :],
            csem.at[CQ + 3],
        )
        cp.start()
        copies.append(cp)

        keep_in.wait()
        keep_out = pltpu.make_async_copy(
            keep_buf, out_ref.at[pl.ds(my_x * m, m), :], keep_out_sem,
        )
        keep_out.start()

        xe.wait_recv()
        y2.wait_recv()
        z2.wait_recv()
        for d in xd + yd + zd:
            d.wait_send()
        xe.wait_send()
        y2.wait_send()
        z2.wait_send()
        for cp in copies:
            cp.wait()
        keep_out.wait()

    return pl.pallas_call(
        body,
        out_shape=jax.ShapeDtypeStruct((out_m, half), x.dtype),
        in_specs=[pl.BlockSpec(memory_space=pl.ANY)],
        out_specs=pl.BlockSpec(memory_space=pl.ANY),
        scratch_shapes=[
            pltpu.VMEM((RQ + A, half), x.dtype),
            pltpu.VMEM((RQ, half), x.dtype),
            pltpu.VMEM((RQ, half), x.dtype),
            pltpu.VMEM((RQ, half), x.dtype),
            pltpu.VMEM((m, half), x.dtype),
            pltpu.SemaphoreType.DMA,
            pltpu.SemaphoreType.DMA,
            pltpu.SemaphoreType.DMA((2,)),
            pltpu.SemaphoreType.DMA((CQ + 1,)),
            pltpu.SemaphoreType.DMA((CQ + 1,)),
            pltpu.SemaphoreType.DMA((CQ + 1,)),
            pltpu.SemaphoreType.DMA((CQ + 1,)),
            pltpu.SemaphoreType.DMA((CQ + 1,)),
            pltpu.SemaphoreType.DMA((CQ + 1,)),
            pltpu.SemaphoreType.DMA((3 * CQ,)),
        ],
        compiler_params=pltpu.CompilerParams(
            collective_id=0, vmem_limit_bytes=64 * 1024 * 1024,
        ),
    )(x)


# device time: 105034 ns/iter; 5.1131x vs baseline; 1.0094x over previous
import jax
import jax.numpy as jnp
from jax import lax
from jax.experimental import pallas as pl
from jax.experimental.pallas import tpu as pltpu

RQ = 1024
CHUNKS = ((0, 64), (64, 192), (256, 256), (512, 256), (768, 256))
NC = len(CHUNKS)
A = 400
B = 312
C2 = RQ - A - B


def kernel(x):
    m, n = x.shape
    half = n // 2
    out_m = 2 * m

    def body(x_ref, out_ref, sbuf, xrbuf, yrbuf, zrbuf, keep_buf,
             keep_sem, keep_out_sem, stage_sems,
             xsend, xrecv, ysend, yrecv, zsend, zrecv, csem):
        my_x = lax.axis_index("x")
        my_y = lax.axis_index("y")
        my_z = lax.axis_index("z")
        xp = (1 - my_x, my_y, my_z)
        yp = (my_x, 1 - my_y, my_z)
        zp = (my_x, my_y, 1 - my_z)
        q = 2 * my_y + my_z
        qy = 2 * (1 - my_y) + my_z
        qz = 2 * my_y + (1 - my_z)
        qd = 2 * (1 - my_y) + (1 - my_z)

        ocol = (1 - my_x) * half
        obase = my_x * m
        rbase = (1 - my_x) * m

        def rdma(src, dst, ssem, rsem, dev):
            return pltpu.make_async_remote_copy(
                src_ref=src, dst_ref=dst, send_sem=ssem, recv_sem=rsem,
                device_id=dev, device_id_type=pl.DeviceIdType.MESH,
            )

        barrier = pltpu.get_barrier_semaphore()
        for nb in (xp, yp, zp):
            pl.semaphore_signal(
                barrier, inc=1, device_id=nb,
                device_id_type=pl.DeviceIdType.MESH,
            )
        pl.semaphore_wait(barrier, 3)

        keep_in = pltpu.make_async_copy(
            x_ref.at[:, pl.ds(my_x * half, half)], keep_buf, keep_sem,
        )
        keep_in.start()

        st0 = pltpu.make_async_copy(
            x_ref.at[pl.ds(q * RQ, RQ), pl.ds(ocol, half)],
            sbuf.at[pl.ds(0, RQ), :], stage_sems.at[0],
        )
        st1 = pltpu.make_async_copy(
            x_ref.at[pl.ds(qd * RQ, A), pl.ds(ocol, half)],
            sbuf.at[pl.ds(RQ, A), :], stage_sems.at[1],
        )
        st0.start()
        st1.start()
        st0.wait()

        xd = []
        for k, (off, rows) in enumerate(CHUNKS):
            d = rdma(sbuf.at[pl.ds(off, rows), :],
                     xrbuf.at[pl.ds(off, rows), :],
                     xsend.at[k], xrecv.at[k], xp)
            d.start()
            xd.append(d)
        st1.wait()
        xe = rdma(sbuf.at[pl.ds(RQ, A), :],
                  out_ref.at[pl.ds(obase + qd * RQ, A), :],
                  xsend.at[NC], xrecv.at[NC], xp)
        xe.start()

        yd = []
        zd = []
        copies = []
        for k, (off, rows) in enumerate(CHUNKS):
            xd[k].wait_recv()
            dy = rdma(xrbuf.at[pl.ds(off, rows), :],
                      yrbuf.at[pl.ds(off, rows), :],
                      ysend.at[k], yrecv.at[k], yp)
            dy.start()
            yd.append(dy)
            dz = rdma(xrbuf.at[pl.ds(off, rows), :],
                      zrbuf.at[pl.ds(off, rows), :],
                      zsend.at[k], zrecv.at[k], zp)
            dz.start()
            zd.append(dz)
            cp = pltpu.make_async_copy(
                xrbuf.at[pl.ds(off, rows), :],
                out_ref.at[pl.ds(rbase + q * RQ + off, rows), :],
                csem.at[k],
            )
            cp.start()
            copies.append(cp)

        for k in range(NC - 1):
            off, rows = CHUNKS[k]
            zd[k].wait_recv()
            cp = pltpu.make_async_copy(
                zrbuf.at[pl.ds(off, rows), :],
                out_ref.at[pl.ds(rbase + qz * RQ + off, rows), :],
                csem.at[NC + k],
            )
            cp.start()
            copies.append(cp)
        y2 = rdma(zrbuf.at[pl.ds(A, B), :],
                  out_ref.at[pl.ds(rbase + qz * RQ + A, B), :],
                  ysend.at[NC], yrecv.at[NC], yp)
        y2.start()

        for k, (off, rows) in enumerate(CHUNKS):
            yd[k].wait_recv()
            cp = pltpu.make_async_copy(
                yrbuf.at[pl.ds(off, rows), :],
                out_ref.at[pl.ds(rbase + qy * RQ + off, rows), :],
                csem.at[2 * NC + k],
            )
            cp.start()
            copies.append(cp)
        z2 = rdma(yrbuf.at[pl.ds(A + B, C2), :],
                  out_ref.at[pl.ds(rbase + qy * RQ + A + B, C2), :],
                  zsend.at[NC], zrecv.at[NC], zp)
        z2.start()

        off, rows = CHUNKS[NC - 1]
        zd[NC - 1].wait_recv()
        cp = pltpu.make_async_copy(
            zrbuf.at[pl.ds(off, rows), :],
            out_ref.at[pl.ds(rbase + qz * RQ + off, rows), :],
            csem.at[NC + NC - 1],
        )
        cp.start()
        copies.append(cp)

        keep_in.wait()
        keep_out = pltpu.make_async_copy(
            keep_buf, out_ref.at[pl.ds(my_x * m, m), :], keep_out_sem,
        )
        keep_out.start()

        xe.wait_recv()
        y2.wait_recv()
        z2.wait_recv()
        for d in xd + yd + zd:
            d.wait_send()
        xe.wait_send()
        y2.wait_send()
        z2.wait_send()
        for cp in copies:
            cp.wait()
        keep_out.wait()

    return pl.pallas_call(
        body,
        out_shape=jax.ShapeDtypeStruct((out_m, half), x.dtype),
        in_specs=[pl.BlockSpec(memory_space=pl.ANY)],
        out_specs=pl.BlockSpec(memory_space=pl.ANY),
        scratch_shapes=[
            pltpu.VMEM((RQ + A, half), x.dtype),
            pltpu.VMEM((RQ, half), x.dtype),
            pltpu.VMEM((RQ, half), x.dtype),
            pltpu.VMEM((RQ, half), x.dtype),
            pltpu.VMEM((m, half), x.dtype),
            pltpu.SemaphoreType.DMA,
            pltpu.SemaphoreType.DMA,
            pltpu.SemaphoreType.DMA((2,)),
            pltpu.SemaphoreType.DMA((NC + 1,)),
            pltpu.SemaphoreType.DMA((NC + 1,)),
            pltpu.SemaphoreType.DMA((NC + 1,)),
            pltpu.SemaphoreType.DMA((NC + 1,)),
            pltpu.SemaphoreType.DMA((NC + 1,)),
            pltpu.SemaphoreType.DMA((NC + 1,)),
            pltpu.SemaphoreType.DMA((3 * NC,)),
        ],
        compiler_params=pltpu.CompilerParams(
            collective_id=0, vmem_limit_bytes=64 * 1024 * 1024,
        ),
    )(x)
